# baseline (speedup 1.0000x reference)
import jax
import jax.numpy as jnp
from jax import lax
from jax.experimental import pallas as pl
from jax.experimental.pallas import tpu as pltpu
from jax.experimental.pallas import tpu_sc as plsc

B, C, H, W = 4, 96, 224, 224
N = H * W
NC, NS, L = 2, 16, 16
CHP = 2 * N // NS
CHP1 = 896
CH2 = 3584
NCHUNK = N // CH2
CPW = C // NS


def _floor_meta(g, size):
    t = g.astype(jnp.int32)
    tf = t.astype(jnp.float32)
    c0 = jnp.where(g < tf, t - 1, t)
    w1 = g - c0.astype(jnp.float32)
    w0 = 1.0 - w1
    c1 = c0 + 1
    v0 = (c0 >= 0) & (c0 <= size - 1)
    v1 = (c1 >= 0) & (c1 <= size - 1)
    w0 = jnp.where(v0, w0, 0.0)
    w1 = jnp.where(v1, w1, 0.0)
    c0 = jnp.clip(c0, 0, size - 1)
    c1 = jnp.clip(c1, 0, size - 1)
    return c0, c1, w0, w1


def _sc_body(xf, gxf, gyf, out, plane_v, mbi, mbw, obuf, p1gx, p1gy,
             p1mi, p1mw, midx_sh, mw_sh):
    c = lax.axis_index("c")
    s = lax.axis_index("s")

    b_loc = s // 8
    b = 2 * c + b_loc
    base = b * N + (s % 8) * CHP
    col0 = b_loc * N + (s % 8) * CHP

    @pl.loop(0, CHP // CHP1)
    def _(j):
        pltpu.sync_copy(gxf.at[pl.ds(base + j * CHP1, CHP1)], p1gx)
        pltpu.sync_copy(gyf.at[pl.ds(base + j * CHP1, CHP1)], p1gy)

        @pl.loop(0, CHP1 // L)
        def _(k):
            sl = pl.ds(k * L, L)
            gx = p1gx[sl]
            gy = p1gy[sl]
            x0, x1, wx0, wx1 = _floor_meta(gx, W)
            y0, y1, wy0, wy1 = _floor_meta(gy, H)
            p1mi[0, sl] = y0 * W + x0
            p1mi[1, sl] = y1 * W + x1
            p1mw[0, sl] = wx0 * wy0
            p1mw[1, sl] = wx1 * wy0
            p1mw[2, sl] = wx0 * wy1
            p1mw[3, sl] = wx1 * wy1

        pltpu.sync_copy(p1mi, midx_sh.at[:, pl.ds(col0 + j * CHP1, CHP1)])
        pltpu.sync_copy(p1mw, mw_sh.at[:, pl.ds(col0 + j * CHP1, CHP1)])

    plsc.subcore_barrier()

    @pl.loop(0, 2 * CPW)
    def _(t):
        bb_loc = t // CPW
        ch = s * CPW + (t % CPW)
        bb = 2 * c + bb_loc
        plane_off = (bb * C + ch) * N
        pltpu.sync_copy(xf.at[pl.ds(plane_off, N)], plane_v)

        @pl.loop(0, NCHUNK)
        def _(q):
            pltpu.sync_copy(
                midx_sh.at[:, pl.ds(bb_loc * N + q * CH2, CH2)], mbi)
            pltpu.sync_copy(
                mw_sh.at[:, pl.ds(bb_loc * N + q * CH2, CH2)], mbw)

            @pl.loop(0, CH2 // L)
            def _(k):
                sl = pl.ds(k * L, L)
                i00 = mbi[0, sl]
                i11 = mbi[1, sl]
                dx = (i11 - i00) & 1
                g00 = plsc.load_gather(plane_v, [i00])
                g01 = plsc.load_gather(plane_v, [i00 + dx])
                g10 = plsc.load_gather(plane_v, [i11 - dx])
                g11 = plsc.load_gather(plane_v, [i11])
                obuf[sl] = (g00 * mbw[0, sl] + g01 * mbw[1, sl]
                            + g10 * mbw[2, sl] + g11 * mbw[3, sl])

            pltpu.sync_copy(obuf, out.at[pl.ds(plane_off + q * CH2, CH2)])


def _sc_call(xf, gxf, gyf):
    mesh = plsc.VectorSubcoreMesh(core_axis_name="c", subcore_axis_name="s",
                                  num_cores=NC, num_subcores=NS)
    f = pl.kernel(
        _sc_body,
        out_type=jax.ShapeDtypeStruct((B * C * N,), jnp.float32),
        mesh=mesh,
        compiler_params=pltpu.CompilerParams(needs_layout_passes=False),
        scratch_types=[
            pltpu.VMEM((N,), jnp.float32),
            pltpu.VMEM((2, CH2), jnp.int32),
            pltpu.VMEM((4, CH2), jnp.float32),
            pltpu.VMEM((CH2,), jnp.float32),
            pltpu.VMEM((CHP1,), jnp.float32),
            pltpu.VMEM((CHP1,), jnp.float32),
            pltpu.VMEM((2, CHP1), jnp.int32),
            pltpu.VMEM((4, CHP1), jnp.float32),
            pltpu.VMEM_SHARED((2, 2 * N), jnp.int32),
            pltpu.VMEM_SHARED((4, 2 * N), jnp.float32),
        ],
    )
    return f(xf, gxf, gyf)


def kernel(offsets, x):
    offs = offsets.reshape(-1, H, W, 2)
    gy_l, gx_l = jnp.meshgrid(jnp.linspace(-1.0, 1.0, H, dtype=x.dtype),
                              jnp.linspace(-1.0, 1.0, W, dtype=x.dtype),
                              indexing="ij")
    grid = jnp.stack([gx_l, gy_l], axis=-1)
    grid = jnp.broadcast_to(grid[None], (B, H, W, 2))
    offn = offs / jnp.array([W, H], dtype=x.dtype).reshape(1, 1, 1, 2) * 2.0
    ng = grid + offn
    gx = (ng[..., 0] + 1.0) * 0.5 * (W - 1)
    gy = (ng[..., 1] + 1.0) * 0.5 * (H - 1)

    out = _sc_call(x.reshape(B * C * N),
                   gx.reshape(B * N), gy.reshape(B * N))
    return out.reshape(B, C, H, W)


# speedup vs baseline: 2.3638x; 2.3638x over previous
import jax
import jax.numpy as jnp
from jax import lax
from jax.experimental import pallas as pl
from jax.experimental.pallas import tpu as pltpu
from jax.experimental.pallas import tpu_sc as plsc

B, C, H, W = 4, 96, 224, 224
N = H * W
NC, NS, L = 2, 16, 16
CHP = 2 * N // NS
CHP1 = 896
CH2 = 3584
NCHUNK = N // CH2
CPW = C // NS


def _floor_meta(g, size):
    t = g.astype(jnp.int32)
    tf = t.astype(jnp.float32)
    c0 = jnp.where(g < tf, t - 1, t)
    w1 = g - c0.astype(jnp.float32)
    w0 = 1.0 - w1
    c1 = c0 + 1
    v0 = (c0 >= 0) & (c0 <= size - 1)
    v1 = (c1 >= 0) & (c1 <= size - 1)
    w0 = jnp.where(v0, w0, 0.0)
    w1 = jnp.where(v1, w1, 0.0)
    c0 = jnp.clip(c0, 0, size - 1)
    c1 = jnp.clip(c1, 0, size - 1)
    return c0, c1, w0, w1


def _sc_body(xf, gxf, gyf, out, plane_v, mbi, mbw, obuf, p1gx, p1gy,
             p1mi, p1mw, midx_sh, mw_sh, sem_m0, sem_m1, sem_o0, sem_o1):
    c = lax.axis_index("c")
    s = lax.axis_index("s")

    b_loc = s // 8
    b = 2 * c + b_loc
    base = b * N + (s % 8) * CHP
    col0 = b_loc * N + (s % 8) * CHP

    @pl.loop(0, CHP // CHP1)
    def _(j):
        pltpu.sync_copy(gxf.at[pl.ds(base + j * CHP1, CHP1)], p1gx)
        pltpu.sync_copy(gyf.at[pl.ds(base + j * CHP1, CHP1)], p1gy)

        @pl.loop(0, CHP1 // L)
        def _(k):
            sl = pl.ds(k * L, L)
            gx = p1gx[sl]
            gy = p1gy[sl]
            x0, x1, wx0, wx1 = _floor_meta(gx, W)
            y0, y1, wy0, wy1 = _floor_meta(gy, H)
            p1mi[0, sl] = y0 * W + x0
            p1mi[1, sl] = y1 * W + x1
            p1mw[0, sl] = plsc.bitcast(
                plsc.pack(wx0 * wy0, wx1 * wy0,
                          format=plsc.PackFormat.INTERLEAVED), jnp.int32)
            p1mw[1, sl] = plsc.bitcast(
                plsc.pack(wx0 * wy1, wx1 * wy1,
                          format=plsc.PackFormat.INTERLEAVED), jnp.int32)

        pltpu.sync_copy(p1mi, midx_sh.at[:, pl.ds(col0 + j * CHP1, CHP1)])
        pltpu.sync_copy(p1mw, mw_sh.at[:, pl.ds(col0 + j * CHP1, CHP1)])

    plsc.subcore_barrier()

    sem_m = (sem_m0, sem_m1)
    sem_o = (sem_o0, sem_o1)

    @pl.loop(0, 2 * CPW)
    def _(t):
        bb_loc = t // CPW
        ch = s * CPW + (t % CPW)
        bb = 2 * c + bb_loc
        plane_off = (bb * C + ch) * N
        mcol = bb_loc * N

        def m_start(q, i):
            pltpu.async_copy(midx_sh.at[:, pl.ds(mcol + q * CH2, CH2)],
                             mbi.at[i], sem_m[i])
            pltpu.async_copy(mw_sh.at[:, pl.ds(mcol + q * CH2, CH2)],
                             mbw.at[i], sem_m[i])

        def m_wait(i):
            pltpu.make_async_copy(midx_sh.at[:, pl.ds(0, CH2)],
                                  mbi.at[i], sem_m[i]).wait()
            pltpu.make_async_copy(mw_sh.at[:, pl.ds(0, CH2)],
                                  mbw.at[i], sem_m[i]).wait()

        def o_start(q, i):
            pltpu.async_copy(obuf.at[i],
                             out.at[pl.ds(plane_off + q * CH2, CH2)],
                             sem_o[i])

        def o_drain(i):
            pltpu.make_async_copy(obuf.at[i],
                                  out.at[pl.ds(plane_off, CH2)],
                                  sem_o[i]).wait()

        def compute(i):
            @plsc.parallel_loop(0, CH2 // L, unroll=4)
            def _(k):
                sl = pl.ds(k * L, L)
                sl2 = pl.ds(k * 2 * L, 2 * L)
                i00 = mbi[i, 0, sl]
                i11 = mbi[i, 1, sl]
                dx = (i11 - i00) & 1
                g00 = plsc.load_gather(plane_v, [i00])
                g01 = plsc.load_gather(plane_v, [i00 + dx])
                g10 = plsc.load_gather(plane_v, [i11 - dx])
                g11 = plsc.load_gather(plane_v, [i11])
                w00, w01 = plsc.unpack(
                    plsc.bitcast(mbw[i, 0, sl], jnp.bfloat16),
                    format=plsc.PackFormat.INTERLEAVED)
                w10, w11 = plsc.unpack(
                    plsc.bitcast(mbw[i, 1, sl], jnp.bfloat16),
                    format=plsc.PackFormat.INTERLEAVED)
                obuf[i, sl] = g00 * w00 + g01 * w01 + g10 * w10 + g11 * w11

        m_start(0, 0)
        pltpu.sync_copy(xf.at[pl.ds(plane_off, N)], plane_v)

        @pl.loop(0, NCHUNK // 2)
        def _(g):
            q0 = 2 * g
            m_start(q0 + 1, 1)
            m_wait(0)

            @pl.when(g > 0)
            def _():
                o_drain(0)

            compute(0)
            o_start(q0, 0)

            @pl.when(g < NCHUNK // 2 - 1)
            def _():
                m_start(q0 + 2, 0)

            m_wait(1)

            @pl.when(g > 0)
            def _():
                o_drain(1)

            compute(1)
            o_start(q0 + 1, 1)

        o_drain(0)
        o_drain(1)


def _sc_call(xf, gxf, gyf):
    mesh = plsc.VectorSubcoreMesh(core_axis_name="c", subcore_axis_name="s",
                                  num_cores=NC, num_subcores=NS)
    f = pl.kernel(
        _sc_body,
        out_type=jax.ShapeDtypeStruct((B * C * N,), jnp.float32),
        mesh=mesh,
        compiler_params=pltpu.CompilerParams(needs_layout_passes=False),
        scratch_types=[
            pltpu.VMEM((N,), jnp.float32),
            pltpu.VMEM((2, 2, CH2), jnp.int32),
            pltpu.VMEM((2, 2, CH2), jnp.int32),
            pltpu.VMEM((2, CH2), jnp.float32),
            pltpu.VMEM((CHP1,), jnp.float32),
            pltpu.VMEM((CHP1,), jnp.float32),
            pltpu.VMEM((2, CHP1), jnp.int32),
            pltpu.VMEM((2, CHP1), jnp.int32),
            pltpu.VMEM_SHARED((2, 2 * N), jnp.int32),
            pltpu.VMEM_SHARED((2, 2 * N), jnp.int32),
            pltpu.SemaphoreType.DMA,
            pltpu.SemaphoreType.DMA,
            pltpu.SemaphoreType.DMA,
            pltpu.SemaphoreType.DMA,
        ],
    )
    return f(xf, gxf, gyf)


def kernel(offsets, x):
    offs = offsets.reshape(-1, H, W, 2)
    gy_l, gx_l = jnp.meshgrid(jnp.linspace(-1.0, 1.0, H, dtype=x.dtype),
                              jnp.linspace(-1.0, 1.0, W, dtype=x.dtype),
                              indexing="ij")
    grid = jnp.stack([gx_l, gy_l], axis=-1)
    grid = jnp.broadcast_to(grid[None], (B, H, W, 2))
    offn = offs / jnp.array([W, H], dtype=x.dtype).reshape(1, 1, 1, 2) * 2.0
    ng = grid + offn
    gx = (ng[..., 0] + 1.0) * 0.5 * (W - 1)
    gy = (ng[..., 1] + 1.0) * 0.5 * (H - 1)

    out = _sc_call(x.reshape(B * C * N),
                   gx.reshape(B * N), gy.reshape(B * N))
    return out.reshape(B, C, H, W)


# speedup vs baseline: 3.1533x; 1.3340x over previous
import jax
import jax.numpy as jnp
from jax import lax
from jax.experimental import pallas as pl
from jax.experimental.pallas import tpu as pltpu
from jax.experimental.pallas import tpu_sc as plsc

B, C, H, W = 4, 96, 224, 224
N = H * W
NC, NS, L = 2, 16, 16
CHP = 2 * N // NS
CHP1 = 896
CH2 = 3584
RPC = CH2 // W
NCHUNK = N // CH2
CPW = C // NS


def _floor_meta(g, size):
    t = g.astype(jnp.int32)
    tf = t.astype(jnp.float32)
    c0 = jnp.where(g < tf, t - 1, t)
    w1 = g - c0.astype(jnp.float32)
    w0 = 1.0 - w1
    c1 = c0 + 1
    v0 = (c0 >= 0) & (c0 <= size - 1)
    v1 = (c1 >= 0) & (c1 <= size - 1)
    w0 = jnp.where(v0, w0, 0.0)
    w1 = jnp.where(v1, w1, 0.0)
    c0 = jnp.clip(c0, 0, size - 1)
    c1 = jnp.clip(c1, 0, size - 1)
    return c0, c1, w0, w1


def _sc_body(xf, gxf, gyf, out, plane_v, mbi, mbw, obuf, p1gx, p1gy,
             p1mi, p1mw, midx_sh, mw_sh, sem_m0, sem_m1, sem_o0, sem_o1):
    c = lax.axis_index("c")
    s = lax.axis_index("s")

    b_loc = s // 8
    b = 2 * c + b_loc
    base = b * N + (s % 8) * CHP
    col0 = b_loc * N + (s % 8) * CHP

    @pl.loop(0, CHP // CHP1)
    def _(j):
        pltpu.sync_copy(gxf.at[pl.ds(base + j * CHP1, CHP1)], p1gx)
        pltpu.sync_copy(gyf.at[pl.ds(base + j * CHP1, CHP1)], p1gy)

        @pl.loop(0, CHP1 // L)
        def _(k):
            sl = pl.ds(k * L, L)
            gx = p1gx[sl]
            gy = p1gy[sl]
            x0, x1, wx0, wx1 = _floor_meta(gx, W)
            y0, y1, wy0, wy1 = _floor_meta(gy, H)
            p1mi[0, sl] = y0 * 256 + x0
            p1mi[1, sl] = y1 * 256 + x1
            p1mw[0, sl] = plsc.bitcast(
                plsc.pack(wx0 * wy0, wx1 * wy0,
                          format=plsc.PackFormat.INTERLEAVED), jnp.int32)
            p1mw[1, sl] = plsc.bitcast(
                plsc.pack(wx0 * wy1, wx1 * wy1,
                          format=plsc.PackFormat.INTERLEAVED), jnp.int32)

        pltpu.sync_copy(p1mi, midx_sh.at[:, pl.ds(col0 + j * CHP1, CHP1)])
        pltpu.sync_copy(p1mw, mw_sh.at[:, pl.ds(col0 + j * CHP1, CHP1)])

    plsc.subcore_barrier()

    sem_m = (sem_m0, sem_m1)
    sem_o = (sem_o0, sem_o1)

    @pl.loop(0, 2 * CPW)
    def _(t):
        bb_loc = t // CPW
        ch = s * CPW + (t % CPW)
        bb = 2 * c + bb_loc
        row0 = (bb * C + ch) * H
        mcol = bb_loc * N

        def m_start(q, i):
            pltpu.async_copy(midx_sh.at[:, pl.ds(mcol + q * CH2, CH2)],
                             mbi.at[i], sem_m[i])
            pltpu.async_copy(mw_sh.at[:, pl.ds(mcol + q * CH2, CH2)],
                             mbw.at[i], sem_m[i])

        def m_wait(i):
            pltpu.make_async_copy(midx_sh.at[:, pl.ds(0, CH2)],
                                  mbi.at[i], sem_m[i]).wait()
            pltpu.make_async_copy(mw_sh.at[:, pl.ds(0, CH2)],
                                  mbw.at[i], sem_m[i]).wait()

        def o_start(q, i):
            pltpu.async_copy(obuf.at[i],
                             out.at[pl.ds(row0 + q * RPC, RPC), :],
                             sem_o[i])

        def o_drain(i):
            pltpu.make_async_copy(obuf.at[i],
                                  out.at[pl.ds(row0, RPC), :],
                                  sem_o[i]).wait()

        def compute(i):
            @pl.loop(0, RPC)
            def _(r):
                @plsc.parallel_loop(0, W // L, unroll=2)
                def _(v):
                    sl = pl.ds(r * W + v * L, L)
                    i00 = mbi[i, 0, sl]
                    i11 = mbi[i, 1, sl]
                    y0 = i00 >> 8
                    x0 = i00 & 255
                    y1 = i11 >> 8
                    x1 = i11 & 255
                    g00 = plsc.load_gather(plane_v, [y0, x0])
                    g01 = plsc.load_gather(plane_v, [y0, x1])
                    g10 = plsc.load_gather(plane_v, [y1, x0])
                    g11 = plsc.load_gather(plane_v, [y1, x1])
                    w00, w01 = plsc.unpack(
                        plsc.bitcast(mbw[i, 0, sl], jnp.bfloat16),
                        format=plsc.PackFormat.INTERLEAVED)
                    w10, w11 = plsc.unpack(
                        plsc.bitcast(mbw[i, 1, sl], jnp.bfloat16),
                        format=plsc.PackFormat.INTERLEAVED)
                    obuf[i, r, pl.ds(v * L, L)] = (
                        g00 * w00 + g01 * w01 + g10 * w10 + g11 * w11)

        m_start(0, 0)
        pltpu.sync_copy(xf.at[pl.ds(row0, H), :], plane_v)

        @pl.loop(0, NCHUNK // 2)
        def _(g):
            q0 = 2 * g
            m_start(q0 + 1, 1)
            m_wait(0)

            @pl.when(g > 0)
            def _():
                o_drain(0)

            compute(0)
            o_start(q0, 0)

            @pl.when(g < NCHUNK // 2 - 1)
            def _():
                m_start(q0 + 2, 0)

            m_wait(1)

            @pl.when(g > 0)
            def _():
                o_drain(1)

            compute(1)
            o_start(q0 + 1, 1)

        o_drain(0)
        o_drain(1)


def _sc_call(xf, gxf, gyf):
    mesh = plsc.VectorSubcoreMesh(core_axis_name="c", subcore_axis_name="s",
                                  num_cores=NC, num_subcores=NS)
    f = pl.kernel(
        _sc_body,
        out_type=jax.ShapeDtypeStruct((B * C * H, W), jnp.float32),
        mesh=mesh,
        compiler_params=pltpu.CompilerParams(needs_layout_passes=False),
        scratch_types=[
            pltpu.VMEM((H, W), jnp.float32),
            pltpu.VMEM((2, 2, CH2), jnp.int32),
            pltpu.VMEM((2, 2, CH2), jnp.int32),
            pltpu.VMEM((2, RPC, W), jnp.float32),
            pltpu.VMEM((CHP1,), jnp.float32),
            pltpu.VMEM((CHP1,), jnp.float32),
            pltpu.VMEM((2, CHP1), jnp.int32),
            pltpu.VMEM((2, CHP1), jnp.int32),
            pltpu.VMEM_SHARED((2, 2 * N), jnp.int32),
            pltpu.VMEM_SHARED((2, 2 * N), jnp.int32),
            pltpu.SemaphoreType.DMA,
            pltpu.SemaphoreType.DMA,
            pltpu.SemaphoreType.DMA,
            pltpu.SemaphoreType.DMA,
        ],
    )
    return f(xf, gxf, gyf)


def kernel(offsets, x):
    offs = offsets.reshape(-1, H, W, 2)
    gy_l, gx_l = jnp.meshgrid(jnp.linspace(-1.0, 1.0, H, dtype=x.dtype),
                              jnp.linspace(-1.0, 1.0, W, dtype=x.dtype),
                              indexing="ij")
    grid = jnp.stack([gx_l, gy_l], axis=-1)
    grid = jnp.broadcast_to(grid[None], (B, H, W, 2))
    offn = offs / jnp.array([W, H], dtype=x.dtype).reshape(1, 1, 1, 2) * 2.0
    ng = grid + offn
    gx = (ng[..., 0] + 1.0) * 0.5 * (W - 1)
    gy = (ng[..., 1] + 1.0) * 0.5 * (H - 1)

    out = _sc_call(x.reshape(B * C * H, W),
                   gx.reshape(B * N), gy.reshape(B * N))
    return out.reshape(B, C, H, W)
